# baseline (speedup 1.0000x reference)
import jax
import jax.numpy as jnp
from jax import lax
from jax.experimental import pallas as pl
from jax.experimental.pallas import tpu as pltpu
from jax.experimental.pallas import tpu_sc as plsc

B = 16384
D = 64
NC = 2
NS = 16
NW = NC * NS
BPW = B // NW
CHUNK = 128
NCHUNK = BPW // CHUNK


def _sc_gather_body(uidx_hbm, iidx_hbm, ug_hbm, ig_hbm, um_hbm, im_hbm,
                    ug_out, ig_out, um_out, im_out,
                    uidx_v, iidx_v, buf0, buf1, sem0, sem1):
  wid = lax.axis_index("s") * NC + lax.axis_index("c")
  base = wid * BPW
  pltpu.sync_copy(uidx_hbm.at[pl.ds(base, BPW)], uidx_v)
  pltpu.sync_copy(iidx_hbm.at[pl.ds(base, BPW)], iidx_v)

  def fire(table, idx_v, buf, sem):
    cps = []
    for j in range(NCHUNK):
      cps.append(pltpu.async_copy(
          table.at[idx_v.at[pl.ds(j * CHUNK, CHUNK)]],
          buf.at[pl.ds(j * CHUNK, CHUNK)], sem))
    return cps

  def drain(cps):
    for cp in cps:
      cp.wait()

  cps0 = fire(ug_hbm, uidx_v, buf0, sem0)
  cps1 = fire(ig_hbm, iidx_v, buf1, sem1)
  drain(cps0)
  pltpu.sync_copy(buf0, ug_out.at[pl.ds(base, BPW)])
  cps0 = fire(um_hbm, uidx_v, buf0, sem0)
  drain(cps1)
  pltpu.sync_copy(buf1, ig_out.at[pl.ds(base, BPW)])
  cps1 = fire(im_hbm, iidx_v, buf1, sem1)
  drain(cps0)
  pltpu.sync_copy(buf0, um_out.at[pl.ds(base, BPW)])
  drain(cps1)
  pltpu.sync_copy(buf1, im_out.at[pl.ds(base, BPW)])


_sc_gather = pl.kernel(
    _sc_gather_body,
    out_type=[jax.ShapeDtypeStruct((B, D), jnp.float32)] * 4,
    mesh=plsc.VectorSubcoreMesh(core_axis_name="c", subcore_axis_name="s"),
    scratch_types=[
        pltpu.VMEM((BPW,), jnp.int32),
        pltpu.VMEM((BPW,), jnp.int32),
        pltpu.VMEM((BPW, D), jnp.float32),
        pltpu.VMEM((BPW, D), jnp.float32),
        pltpu.SemaphoreType.DMA,
        pltpu.SemaphoreType.DMA,
    ],
    compiler_params=pltpu.CompilerParams(use_tc_tiling_on_sc=False),
)

BLK = 2048


def _dense_body(ug_ref, ig_ref, um_ref, im_ref, w1t_ref, b1_ref, wp_ref,
                bp_ref, out_ref):
  gmf = ug_ref[...] * ig_ref[...]
  h = jnp.dot(um_ref[...], w1t_ref[:D, :], preferred_element_type=jnp.float32)
  h = h + jnp.dot(im_ref[...], w1t_ref[D:, :],
                  preferred_element_type=jnp.float32)
  h = jnp.maximum(h + b1_ref[...], 0.0)
  pred = jnp.sum(gmf * wp_ref[:, :D], axis=1)
  pred = pred + jnp.sum(h * wp_ref[:, D:], axis=1)
  out_ref[...] = pred + bp_ref[0, 0]


def _dense_call(ug, ig, um, im, w1t, b1_2d, wp, bp_2d):
  grid = (B // BLK,)
  row_spec = pl.BlockSpec((BLK, D), lambda i: (i, 0))
  return pl.pallas_call(
      _dense_body,
      grid=grid,
      in_specs=[
          row_spec, row_spec, row_spec, row_spec,
          pl.BlockSpec((2 * D, D), lambda i: (0, 0)),
          pl.BlockSpec((1, D), lambda i: (0, 0)),
          pl.BlockSpec((1, 2 * D), lambda i: (0, 0)),
          pl.BlockSpec((1, 1), lambda i: (0, 0)),
      ],
      out_specs=pl.BlockSpec((BLK,), lambda i: (i,)),
      out_shape=jax.ShapeDtypeStruct((B,), jnp.float32),
  )(ug, ig, um, im, w1t, b1_2d, wp, bp_2d)


def kernel(user_indices, item_indices, user_gmf_table, item_gmf_table,
           user_mlp_table, item_mlp_table, W1, b1, Wp, bp):
  ug, ig, um, im = _sc_gather(
      user_indices.astype(jnp.int32), item_indices.astype(jnp.int32),
      user_gmf_table, item_gmf_table, user_mlp_table, item_mlp_table)
  w1t = W1.T
  return _dense_call(ug, ig, um, im, w1t, b1.reshape(1, D), Wp,
                     bp.reshape(1, 1))
